# Initial kernel scaffold; baseline (speedup 1.0000x reference)
import jax
import jax.numpy as jnp
from jax.experimental import pallas as pl


def kernel(x, pos, idx, W_in, Wq, bq, Wk, bk, Wv, bv, pe_w1, pe_b1, pe_w2, pe_b2, mg_w1, mg_b1, mg_w2, mg_b2, W_out):
    raise NotImplementedError("write your pallas kernel here")


# speedup vs baseline: 6.5064x; 6.5064x over previous
import functools

import jax
import jax.numpy as jnp
from jax import lax
from jax.experimental import pallas as pl
from jax.experimental.pallas import tpu as pltpu
from jax.experimental.pallas import tpu_sc as plsc

N = 100000
K = 16
INC = 32
MID = 8
E = N * K
EPS = 1e-5

NT = 32
PPT = N // NT
CHUNK = 125
NCH = PPT // CHUNK
EC = CHUNK * K
_SPANS = [(i * 128, 128) for i in range(15)] + [(1920, 80)]

_BN = 4000
_GRID = N // _BN

f32 = jnp.float32
i32 = jnp.int32

_mesh = plsc.VectorSubcoreMesh(core_axis_name="c", subcore_axis_name="s")
_SC_PARAMS = pltpu.CompilerParams(needs_layout_passes=False,
                                  use_tc_tiling_on_sc=False)


def _wid():
    return lax.axis_index("s") * 2 + lax.axis_index("c")


def _splat_store(buf, addr, val, lane0):
    plsc.store_scatter(buf, [jnp.full((16,), addr, i32)],
                       jnp.full((16,), val, f32), mask=lane0)


@functools.partial(
    pl.kernel, mesh=_mesh, compiler_params=_SC_PARAMS,
    out_type=jax.ShapeDtypeStruct((NT * N,), f32),
    scratch_types=[pltpu.VMEM((N,), f32),
                   pltpu.VMEM((PPT // 5 * K,), i32)])
def _sc_counts(idx_h, out_h, cnt_v, idxbuf):
    wid = _wid()
    zeros16 = jnp.zeros((16,), f32)
    ones16 = jnp.ones((16,), f32)

    def zbody(i, _):
        cnt_v[pl.ds(i * 16, 16)] = zeros16
        return 0
    lax.fori_loop(0, N // 16, zbody, 0)

    csz = PPT // 5 * K

    def cbody(ch, _):
        pltpu.sync_copy(idx_h.at[pl.ds(wid * PPT * K + ch * csz, csz)], idxbuf)

        def sbody(g, _):
            vec = idxbuf[pl.ds(g * 16, 16)]
            plsc.addupdate_scatter(cnt_v, [vec], ones16)
            return 0
        lax.fori_loop(0, csz // 16, sbody, 0)
        return 0
    lax.fori_loop(0, 5, cbody, 0)
    pltpu.sync_copy(cnt_v, out_h.at[pl.ds(wid * N, N)])


_STRIPE = 3200


@functools.partial(
    pl.kernel, mesh=_mesh, compiler_params=_SC_PARAMS,
    out_type=jax.ShapeDtypeStruct((N,), f32),
    scratch_types=[pltpu.VMEM((_STRIPE,), f32),
                   pltpu.VMEM((_STRIPE,), f32)])
def _sc_cntsum(c32_h, out_h, acc, src):
    wid = _wid()
    start = jnp.minimum(wid * _STRIPE, N - _STRIPE)
    zeros16 = jnp.zeros((16,), f32)

    def zbody(g, _):
        acc[pl.ds(g * 16, 16)] = zeros16
        return 0
    lax.fori_loop(0, _STRIPE // 16, zbody, 0)

    def tbody(t, _):
        pltpu.sync_copy(c32_h.at[pl.ds(t * N + start, _STRIPE)], src)

        def gbody(g, _):
            sl = pl.ds(g * 16, 16)
            acc[sl] = acc[sl] + src[sl]
            return 0
        lax.fori_loop(0, _STRIPE // 16, gbody, 0)
        return 0
    lax.fori_loop(0, NT, tbody, 0)
    pltpu.sync_copy(acc, out_h.at[pl.ds(start, _STRIPE)])


def _issue(tbl_h, b_h, wid, idxall, ch, gbuf, bbuf, semg, semb):
    e0r = ch * EC
    for o, s in _SPANS:
        pltpu.async_copy(tbl_h.at[idxall.at[pl.ds(e0r + o, s)]],
                         gbuf.at[pl.ds(o, s)], semg)
    pltpu.async_copy(b_h.at[pl.ds((wid * PPT + ch * CHUNK) * 8, CHUNK * 8)],
                     bbuf.at[pl.ds(0, CHUNK * 8)], semb)


def _drain(tbl_h, b_h, wid, idxall, ch, gbuf, bbuf, semg, semb):
    e0r = ch * EC
    for o, s in _SPANS:
        pltpu.make_async_copy(tbl_h.at[idxall.at[pl.ds(e0r + o, s)]],
                              gbuf.at[pl.ds(o, s)], semg).wait()
    pltpu.make_async_copy(
        b_h.at[pl.ds((wid * PPT + ch * CHUNK) * 8, CHUNK * 8)],
        bbuf.at[pl.ds(0, CHUNK * 8)], semb).wait()


def _pipeline(idx_h, tbl_h, b_h, wid, idxall, g0, g1, b0, b1, sems, consume,
              accs0):
    pltpu.sync_copy(idx_h.at[pl.ds(wid * PPT * K, PPT * K)], idxall)
    _issue(tbl_h, b_h, wid, idxall, 0, g0, b0, sems[0], sems[2])

    def outer(g, accs):
        ca = 2 * g
        _issue(tbl_h, b_h, wid, idxall, ca + 1, g1, b1, sems[1], sems[3])
        _drain(tbl_h, b_h, wid, idxall, ca, g0, b0, sems[0], sems[2])
        accs = consume(ca, g0, b0, accs)
        _issue(tbl_h, b_h, wid, idxall, ca + 2, g0, b0, sems[0], sems[2])
        _drain(tbl_h, b_h, wid, idxall, ca + 1, g1, b1, sems[1], sems[3])
        return consume(ca + 1, g1, b1, accs)

    accs = lax.fori_loop(0, (NCH - 1) // 2, outer, accs0)
    _drain(tbl_h, b_h, wid, idxall, NCH - 1, g0, b0, sems[0], sems[2])
    return consume(NCH - 1, g0, b0, accs)


_PIPE_SCRATCH = [pltpu.VMEM((PPT * K,), i32),
                 pltpu.VMEM((EC, 16), f32),
                 pltpu.VMEM((EC, 16), f32),
                 pltpu.VMEM((CHUNK * 8 + 8,), f32),
                 pltpu.VMEM((CHUNK * 8 + 8,), f32),
                 pltpu.SemaphoreType.DMA,
                 pltpu.SemaphoreType.DMA,
                 pltpu.SemaphoreType.DMA,
                 pltpu.SemaphoreType.DMA]


@functools.partial(
    pl.kernel, mesh=_mesh, compiler_params=_SC_PARAMS,
    out_type=jax.ShapeDtypeStruct((NT * 32,), f32),
    scratch_types=[pltpu.VMEM((32,), f32)] + _PIPE_SCRATCH)
def _sc_p1(tbl_h, idx_h, b_h, out_h, outbuf, idxall, g0, g1, b0, b1,
           sg0, sg1, sb0, sb1):
    wid = _wid()
    iota = lax.iota(i32, 16)
    lane0 = iota == 0
    cols = [jnp.full((16,), c, i32) for c in range(8)]

    def consume(ch, gbuf, bbuf, accs):
        def pbody(p, accs):
            a1, a2, a3 = accs
            rowv = iota + p * K
            bvec = bbuf[pl.ds(p * 8, 16)]
            na1, na2, na3 = [], [], []
            for c in range(8):
                a = plsc.load_gather(gbuf, [rowv, cols[c]])
                bs = jnp.full((16,), bvec[c], f32)
                na1.append(a1[c] + a)
                na2.append(a2[c] + a * a)
                na3.append(a3[c] + a * bs)
            return tuple(na1), tuple(na2), tuple(na3)
        return lax.fori_loop(0, CHUNK, pbody, accs)

    z = tuple(jnp.zeros((16,), f32) for _ in range(8))
    a1, a2, a3 = _pipeline(idx_h, tbl_h, b_h, wid, idxall, g0, g1, b0, b1,
                           (sg0, sg1, sb0, sb1), consume, (z, z, z))
    for c in range(8):
        _splat_store(outbuf, c, jnp.sum(a1[c]), lane0)
        _splat_store(outbuf, 8 + c, jnp.sum(a2[c]), lane0)
        _splat_store(outbuf, 16 + c, jnp.sum(a3[c]), lane0)
        _splat_store(outbuf, 24 + c, 0.0, lane0)
    pltpu.sync_copy(outbuf, out_h.at[pl.ds(wid * 32, 32)])


_PAIRS = [(ci, cj) for ci in range(8) for cj in range(ci, 8)]


@functools.partial(
    pl.kernel, mesh=_mesh, compiler_params=_SC_PARAMS,
    out_type=jax.ShapeDtypeStruct((NT * 48,), f32),
    scratch_types=[pltpu.VMEM((48,), f32)] + _PIPE_SCRATCH)
def _sc_p2(tbl_h, idx_h, bt_h, out_h, outbuf, idxall, g0, g1, b0, b1,
           sg0, sg1, sb0, sb1):
    wid = _wid()
    iota = lax.iota(i32, 16)
    lane0 = iota == 0
    cols = [jnp.full((16,), c, i32) for c in range(8)]

    def consume(ch, gbuf, bbuf, accs):
        def pbody(p, accs):
            m, sr = accs
            rowv = iota + p * K
            bvec = bbuf[pl.ds(p * 8, 16)]
            r = []
            nsr = []
            for c in range(8):
                a = plsc.load_gather(gbuf, [rowv, cols[c]])
                bs = jnp.full((16,), bvec[c], f32)
                rc = jnp.maximum(a - bs, 0.0)
                r.append(rc)
                nsr.append(sr[c] + rc)
            nm = tuple(m[t] + r[ci] * r[cj]
                       for t, (ci, cj) in enumerate(_PAIRS))
            return nm, tuple(nsr)
        return lax.fori_loop(0, CHUNK, pbody, accs)

    zm = tuple(jnp.zeros((16,), f32) for _ in range(36))
    zs = tuple(jnp.zeros((16,), f32) for _ in range(8))
    m, sr = _pipeline(idx_h, tbl_h, bt_h, wid, idxall, g0, g1, b0, b1,
                      (sg0, sg1, sb0, sb1), consume, (zm, zs))
    for t in range(36):
        _splat_store(outbuf, t, jnp.sum(m[t]), lane0)
    for c in range(8):
        _splat_store(outbuf, 36 + c, jnp.sum(sr[c]), lane0)
    for t in range(44, 48):
        _splat_store(outbuf, t, 0.0, lane0)
    pltpu.sync_copy(outbuf, out_h.at[pl.ds(wid * 48, 48)])


@functools.partial(
    pl.kernel, mesh=_mesh, compiler_params=_SC_PARAMS,
    out_type=jax.ShapeDtypeStruct((N * 16,), f32),
    scratch_types=[pltpu.VMEM((CHUNK * 16,), f32),
                   pltpu.VMEM((80,), f32)] + _PIPE_SCRATCH)
def _sc_p3(tbl_h, idx_h, bt_h, w_h, out_h, obuf, wbuf, idxall, g0, g1,
           b0, b1, sg0, sg1, sb0, sb1):
    wid = _wid()
    iota = lax.iota(i32, 16)
    lane0 = iota == 0
    cols = [jnp.full((16,), c, i32) for c in range(16)]
    pltpu.sync_copy(w_h, wbuf)
    wv = [wbuf[pl.ds(i * 16, 16)] for i in range(5)]

    def wsc(j):
        return wv[j // 16][j % 16]

    def consume(ch, gbuf, bbuf, _):
        pt0 = wid * PPT + ch * CHUNK

        def pbody(p, _):
            rowv = iota + p * K
            bvec = bbuf[pl.ds(p * 8, 16)]
            r = []
            for c in range(8):
                a = plsc.load_gather(gbuf, [rowv, cols[c]])
                bs = jnp.full((16,), bvec[c], f32)
                r.append(jnp.maximum(a - bs, 0.0))
            for cp in range(8):
                z = jnp.full((16,), wsc(64 + cp), f32)
                for c in range(8):
                    z = z + r[c] * wsc(c * 8 + cp)
                e = jnp.exp(jnp.maximum(z, 0.0))
                vp = plsc.load_gather(gbuf, [rowv, cols[8 + cp]])
                _splat_store(obuf, p * 16 + cp, jnp.sum(vp * e), lane0)
                _splat_store(obuf, p * 16 + 8 + cp, jnp.sum(e), lane0)
            return 0
        lax.fori_loop(0, CHUNK, pbody, 0)
        pltpu.sync_copy(obuf, out_h.at[pl.ds(pt0 * 16, CHUNK * 16)])
        return 0

    _pipeline(idx_h, tbl_h, bt_h, wid, idxall, g0, g1, b0, b1,
              (sg0, sg1, sb0, sb1), consume, 0)


def _full(shape):
    return pl.BlockSpec(shape, lambda i: tuple(0 for _ in shape))


def _rows(width):
    return pl.BlockSpec((_BN, width), lambda i: (i, 0))


def _tc1_body(x_ref, pos_ref, cnt_ref, win_ref, wq_ref, bq_ref, wk_ref,
              bk_ref, wv_ref, bv_ref, mw1_ref, mb1_ref, pw1_ref, pb1_ref,
              bp_ref, ka_ref, v_ref, p1_ref, s1_ref, s1q_ref,
              sb_ref, sbb_ref):
    xx = jnp.dot(x_ref[...], win_ref[...], preferred_element_type=f32)
    q = jnp.dot(xx, wq_ref[...], preferred_element_type=f32) + bq_ref[...]
    kk = jnp.dot(xx, wk_ref[...], preferred_element_type=f32) + bk_ref[...]
    v = jnp.dot(xx, wv_ref[...], preferred_element_type=f32) + bv_ref[...]
    bp = jnp.dot(q, mw1_ref[...], preferred_element_type=f32) - mb1_ref[...]
    ka = jnp.dot(kk, mw1_ref[...], preferred_element_type=f32)
    pos = pos_ref[...]
    pw1 = pw1_ref[...]
    p1 = pb1_ref[...] + (pos[:, 0:1] * pw1[0:1, :] + pos[:, 1:2] *
                         pw1[1:2, :] + pos[:, 2:3] * pw1[2:3, :])
    c = cnt_ref[...][:, 0]
    bp_ref[...] = bp
    ka_ref[...] = ka
    v_ref[...] = v
    p1_ref[...] = p1

    @pl.when(pl.program_id(0) == 0)
    def _():
        s1_ref[...] = jnp.zeros_like(s1_ref)
        s1q_ref[...] = jnp.zeros_like(s1q_ref)
        sb_ref[...] = jnp.zeros_like(sb_ref)
        sbb_ref[...] = jnp.zeros_like(sbb_ref)
    cw = c[:, None]
    s1_ref[...] += jnp.sum(cw * p1, axis=0)[None, :]
    s1q_ref[...] += jnp.sum(cw * p1 * p1, axis=0)[None, :]
    sb_ref[...] += jnp.sum(bp, axis=0)[None, :]
    sbb_ref[...] += jnp.sum(bp * bp, axis=0)[None, :]


def _tc2_body(p1_ref, cnt_ref, sc_ref, pw2_ref, pb2_ref,
              p2_ref, s2_ref, s2q_ref):
    sc = sc_ref[...]
    m1 = sc[0:1, 0:3]
    i1 = sc[0:1, 3:6]
    r1 = jnp.maximum((p1_ref[...] - m1) * i1, 0.0)
    pw2 = pw2_ref[...]
    p2 = pb2_ref[...] + (r1[:, 0:1] * pw2[0:1, :] + r1[:, 1:2] *
                         pw2[1:2, :] + r1[:, 2:3] * pw2[2:3, :])
    p2_ref[...] = p2

    @pl.when(pl.program_id(0) == 0)
    def _():
        s2_ref[...] = jnp.zeros_like(s2_ref)
        s2q_ref[...] = jnp.zeros_like(s2q_ref)
    cw = cnt_ref[...]
    s2_ref[...] += jnp.sum(cw * p2, axis=0)[None, :]
    s2q_ref[...] += jnp.sum(cw * p2 * p2, axis=0)[None, :]


def _tc3_body(p2_ref, sc_ref, ka_ref, v_ref, mw1_ref, tbl_ref):
    sc = sc_ref[...]
    penc = jnp.maximum((p2_ref[...] - sc[0:1, 0:8]) * sc[0:1, 8:16], 0.0)
    a = ka_ref[...] + jnp.dot(penc, mw1_ref[...], preferred_element_type=f32)
    tbl_ref[...] = jnp.concatenate([a, v_ref[...] + penc], axis=1)


def _tc3b_body(bp_ref, m3_ref, bt_ref):
    bt_ref[...] = bp_ref[...] + m3_ref[...]


def _tc4_body(nd_ref, x_ref, wo_ref, out_ref):
    nd = nd_ref[...]
    s = nd[:, 0:8] / nd[:, 8:16]
    out_ref[...] = x_ref[...] + jnp.dot(s, wo_ref[...],
                                        preferred_element_type=f32)


def kernel(x, pos, idx, W_in, Wq, bq, Wk, bk, Wv, bv, pe_w1, pe_b1, pe_w2,
           pe_b2, mg_w1, mg_b1, mg_w2, mg_b2, W_out):
    idx_flat = idx.reshape(-1)
    Ef = jnp.float32(E)

    c32 = _sc_counts(idx_flat)
    cnt = _sc_cntsum(c32.reshape(-1)).reshape(N, 1)

    tc1 = pl.pallas_call(
        _tc1_body,
        grid=(_GRID,),
        in_specs=[_rows(INC), _rows(3), _rows(1),
                  _full((INC, MID)), _full((MID, MID)), _full((1, MID)),
                  _full((MID, MID)), _full((1, MID)), _full((MID, MID)),
                  _full((1, MID)), _full((MID, MID)), _full((1, MID)),
                  _full((3, 3)), _full((1, 3))],
        out_specs=[_rows(MID), _rows(MID), _rows(MID), _rows(3),
                   _full((1, 3)), _full((1, 3)), _full((1, MID)),
                   _full((1, MID))],
        out_shape=[jax.ShapeDtypeStruct((N, MID), f32),
                   jax.ShapeDtypeStruct((N, MID), f32),
                   jax.ShapeDtypeStruct((N, MID), f32),
                   jax.ShapeDtypeStruct((N, 3), f32),
                   jax.ShapeDtypeStruct((1, 3), f32),
                   jax.ShapeDtypeStruct((1, 3), f32),
                   jax.ShapeDtypeStruct((1, MID), f32),
                   jax.ShapeDtypeStruct((1, MID), f32)])
    bp, ka, vv, p1, s1, s1q, sb, sbb = tc1(
        x, pos, cnt, W_in, Wq, bq.reshape(1, -1), Wk, bk.reshape(1, -1),
        Wv, bv.reshape(1, -1), mg_w1, mg_b1.reshape(1, -1), pe_w1,
        pe_b1.reshape(1, -1))

    m1 = s1[0] / Ef
    v1 = s1q[0] / Ef - m1 * m1
    i1 = lax.rsqrt(v1 + EPS)
    sc12 = jnp.concatenate([m1, i1, jnp.zeros((2,), f32)]).reshape(1, 8)

    tc2 = pl.pallas_call(
        _tc2_body,
        grid=(_GRID,),
        in_specs=[_rows(3), _rows(1),
                  _full((1, 8)), _full((3, MID)), _full((1, MID))],
        out_specs=[_rows(MID), _full((1, MID)), _full((1, MID))],
        out_shape=[jax.ShapeDtypeStruct((N, MID), f32),
                   jax.ShapeDtypeStruct((1, MID), f32),
                   jax.ShapeDtypeStruct((1, MID), f32)])
    p2, s2, s2q = tc2(p1, cnt, sc12, pe_w2, pe_b2.reshape(1, -1))

    m2 = s2[0] / Ef
    v2 = s2q[0] / Ef - m2 * m2
    i2 = lax.rsqrt(v2 + EPS)
    sc23 = jnp.concatenate([m2, i2]).reshape(1, 16)

    tc3 = pl.pallas_call(
        _tc3_body,
        grid=(_GRID,),
        in_specs=[_rows(MID), _full((1, 16)), _rows(MID), _rows(MID),
                  _full((MID, MID))],
        out_specs=_rows(16),
        out_shape=jax.ShapeDtypeStruct((N, 16), f32))
    tbl = tc3(p2, sc23, ka, vv, mg_w1)

    part1 = _sc_p1(tbl, idx_flat, bp.reshape(-1)).reshape(NT, 32)
    p1sums = jnp.sum(part1, axis=0)
    sa, saa, sab = p1sums[0:8], p1sums[8:16], p1sums[16:24]
    m3 = (sa - K * sb[0]) / Ef
    v3 = (saa - 2.0 * sab + K * sbb[0]) / Ef - m3 * m3
    i3 = lax.rsqrt(v3 + EPS)

    tc3b = pl.pallas_call(
        _tc3b_body,
        grid=(_GRID,),
        in_specs=[_rows(MID), _full((1, MID))],
        out_specs=_rows(MID),
        out_shape=jax.ShapeDtypeStruct((N, MID), f32))
    bt = tc3b(bp, m3.reshape(1, MID))

    part2 = jnp.sum(_sc_p2(tbl, idx_flat, bt.reshape(-1)).reshape(NT, 48), axis=0)
    mu = jnp.zeros((8, 8), f32)
    for t, (ci, cj) in enumerate(_PAIRS):
        mu = mu.at[ci, cj].set(part2[t])
        if ci != cj:
            mu = mu.at[cj, ci].set(part2[t])
    srv = part2[36:44]
    w2p = i3[:, None] * mg_w2
    srw = srv @ w2p
    sw2 = srw + Ef * mg_b2
    sw2q = (jnp.einsum("ic,ij,jc->c", w2p, mu, w2p) + 2.0 * mg_b2 * srw +
            Ef * mg_b2 * mg_b2)
    m4 = sw2 / Ef
    v4 = sw2q / Ef - m4 * m4
    i4 = lax.rsqrt(v4 + EPS)
    w2pp = w2p * i4[None, :]
    b2pp = (mg_b2 - m4) * i4
    wpack = jnp.concatenate([w2pp.reshape(-1), b2pp, jnp.zeros((8,), f32)])

    nd = _sc_p3(tbl, idx_flat, bt.reshape(-1), wpack)

    tc4 = pl.pallas_call(
        _tc4_body,
        grid=(_GRID,),
        in_specs=[_rows(16), _rows(INC), _full((MID, INC))],
        out_specs=_rows(INC),
        out_shape=jax.ShapeDtypeStruct((N, INC), f32))
    return tc4(nd.reshape(N, 16), x, W_out)
